# baseline (speedup 1.0000x reference)
import jax
import jax.numpy as jnp
from jax.experimental import pallas as pl

N_CLS = 1000
N_CTX = 4
DIM = 512
CTX_LEN = 77
SUFFIX_LEN = CTX_LEN - 1 - N_CTX

G = 8


def _concat_kernel(pref_ref, ctx_ref, suf_ref, out_ref):
    out_ref[:, 0:1, :] = pref_ref[...]
    ctx = ctx_ref[...]
    out_ref[:, 1:1 + N_CTX, :] = jnp.broadcast_to(ctx[None, :, :], (G, N_CTX, DIM))
    out_ref[:, 1 + N_CTX:, :] = suf_ref[...]


def kernel(prefixs, ctx, suffixs):
    grid = (N_CLS // G,)
    return pl.pallas_call(
        _concat_kernel,
        grid=grid,
        in_specs=[
            pl.BlockSpec((G, 1, DIM), lambda i: (i, 0, 0)),
            pl.BlockSpec((N_CTX, DIM), lambda i: (0, 0)),
            pl.BlockSpec((G, SUFFIX_LEN, DIM), lambda i: (i, 0, 0)),
        ],
        out_specs=pl.BlockSpec((G, CTX_LEN, DIM), lambda i: (i, 0, 0)),
        out_shape=jax.ShapeDtypeStruct((N_CLS, CTX_LEN, DIM), jnp.float32),
    )(prefixs, ctx, suffixs)


# speedup vs baseline: 1.1897x; 1.1897x over previous
import jax
import jax.numpy as jnp
from jax.experimental import pallas as pl
from jax.experimental.pallas import tpu as pltpu

N_CLS = 1000
N_CTX = 4
DIM = 512
CTX_LEN = 77
SUFFIX_LEN = CTX_LEN - 1 - N_CTX

G = 40


def _concat_kernel(pref_ref, ctx_ref, suf_ref, out_ref):
    out_ref[:, 0:1, :] = pref_ref[...]
    ctx = ctx_ref[...]
    out_ref[:, 1:1 + N_CTX, :] = jnp.broadcast_to(ctx[None, :, :], (G, N_CTX, DIM))
    out_ref[:, 1 + N_CTX:, :] = suf_ref[...]


def kernel(prefixs, ctx, suffixs):
    grid = (N_CLS // G,)
    return pl.pallas_call(
        _concat_kernel,
        grid=grid,
        in_specs=[
            pl.BlockSpec((G, 1, DIM), lambda i: (i, 0, 0)),
            pl.BlockSpec((N_CTX, DIM), lambda i: (0, 0)),
            pl.BlockSpec((G, SUFFIX_LEN, DIM), lambda i: (i, 0, 0)),
        ],
        out_specs=pl.BlockSpec((G, CTX_LEN, DIM), lambda i: (i, 0, 0)),
        out_shape=jax.ShapeDtypeStruct((N_CLS, CTX_LEN, DIM), jnp.float32),
        compiler_params=pltpu.CompilerParams(
            dimension_semantics=("arbitrary",),
        ),
    )(prefixs, ctx, suffixs)


# speedup vs baseline: 1.1918x; 1.0018x over previous
import jax
import jax.numpy as jnp
from jax.experimental import pallas as pl
from jax.experimental.pallas import tpu as pltpu

N_CLS = 1000
N_CTX = 4
DIM = 512
CTX_LEN = 77
SUFFIX_LEN = CTX_LEN - 1 - N_CTX

G = 50


def _concat_kernel(pref_ref, ctx_ref, suf_ref, out_ref):
    out_ref[:, 0:1, :] = pref_ref[...]
    ctx = ctx_ref[...]
    out_ref[:, 1:1 + N_CTX, :] = jnp.broadcast_to(ctx[None, :, :], (G, N_CTX, DIM))
    out_ref[:, 1 + N_CTX:, :] = suf_ref[...]


def kernel(prefixs, ctx, suffixs):
    grid = (N_CLS // G,)
    return pl.pallas_call(
        _concat_kernel,
        grid=grid,
        in_specs=[
            pl.BlockSpec((G, 1, DIM), lambda i: (i, 0, 0)),
            pl.BlockSpec((N_CTX, DIM), lambda i: (0, 0)),
            pl.BlockSpec((G, SUFFIX_LEN, DIM), lambda i: (i, 0, 0)),
        ],
        out_specs=pl.BlockSpec((G, CTX_LEN, DIM), lambda i: (i, 0, 0)),
        out_shape=jax.ShapeDtypeStruct((N_CLS, CTX_LEN, DIM), jnp.float32),
        compiler_params=pltpu.CompilerParams(
            dimension_semantics=("arbitrary",),
        ),
    )(prefixs, ctx, suffixs)
